# baseline (speedup 1.0000x reference)
import functools

import jax
import jax.numpy as jnp
from jax import lax
from jax.experimental import pallas as pl
from jax.experimental.pallas import tpu as pltpu
from jax.experimental.pallas import tpu_sc as plsc

B = 16384
DA = 64
DM = 64
DOUT = 128
DPAIR = 128
VH = 50000
NC = 2
NS = 16
NW = NC * NS
B_PER_W = B // NW
CHUNK = 128
NCHUNK = B_PER_W // CHUNK


def _sc_gather(aff_idx_r, mat_idx_r, aff_pairs, mat_pairs):
    mesh = plsc.VectorSubcoreMesh(core_axis_name="c", subcore_axis_name="s")

    @functools.partial(
        pl.kernel,
        mesh=mesh,
        out_type=[
            jax.ShapeDtypeStruct((B, DPAIR), jnp.float32),
            jax.ShapeDtypeStruct((B, DPAIR), jnp.float32),
        ],
        scratch_types=[
            pltpu.VMEM((NCHUNK, CHUNK), jnp.int32),
            pltpu.VMEM((NCHUNK, CHUNK), jnp.int32),
            pltpu.VMEM((B_PER_W, DPAIR), jnp.float32),
            pltpu.SemaphoreType.DMA,
        ],
    )
    def k(aff_idx_hbm, mat_idx_hbm, aff_t_hbm, mat_t_hbm,
          out_a_hbm, out_m_hbm, aidx_v, midx_v, rows_v, sem):
        wid = lax.axis_index("s") * NC + lax.axis_index("c")
        base = wid * B_PER_W
        pltpu.sync_copy(aff_idx_hbm.at[wid], aidx_v)
        pltpu.sync_copy(mat_idx_hbm.at[wid], midx_v)
        for idx_v, t_hbm, out_hbm in (
            (aidx_v, aff_t_hbm, out_a_hbm),
            (midx_v, mat_t_hbm, out_m_hbm),
        ):
            copies = []
            for j in range(NCHUNK):
                copies.append(pltpu.async_copy(
                    t_hbm.at[idx_v.at[j]],
                    rows_v.at[pl.ds(j * CHUNK, CHUNK)], sem))
            for c in copies:
                c.wait()
            pltpu.sync_copy(rows_v, out_hbm.at[pl.ds(base, B_PER_W)])

    return k(aff_idx_r, mat_idx_r, aff_pairs, mat_pairs)


_BT = 2048


def _tc_body(a_ref, m_ref, pa_ref, pm_ref, wa_ref, wm_ref, b_ref, out_ref):
    lane = lax.broadcasted_iota(jnp.int32, (_BT, DPAIR), 1)
    left = lane < DA
    mask_a = jnp.where(left, 1.0 - pa_ref[...], pa_ref[...])
    mask_m = jnp.where(left, 1.0 - pm_ref[...], pm_ref[...])
    acc = jnp.dot(a_ref[...] * mask_a, wa_ref[...],
                  preferred_element_type=jnp.float32)
    acc += jnp.dot(m_ref[...] * mask_m, wm_ref[...],
                   preferred_element_type=jnp.float32)
    out_ref[...] = jnp.maximum(acc + b_ref[...], 0.0)


def _tc_linear(aff_e, mat_e, pa, pm, wa2, wm2, b2d):
    grid = (B // _BT,)
    return pl.pallas_call(
        _tc_body,
        grid=grid,
        in_specs=[
            pl.BlockSpec((_BT, DPAIR), lambda i: (i, 0)),
            pl.BlockSpec((_BT, DPAIR), lambda i: (i, 0)),
            pl.BlockSpec((_BT, 1), lambda i: (i, 0)),
            pl.BlockSpec((_BT, 1), lambda i: (i, 0)),
            pl.BlockSpec((DPAIR, DOUT), lambda i: (0, 0)),
            pl.BlockSpec((DPAIR, DOUT), lambda i: (0, 0)),
            pl.BlockSpec((1, DOUT), lambda i: (0, 0)),
        ],
        out_specs=pl.BlockSpec((_BT, DOUT), lambda i: (i, 0)),
        out_shape=jax.ShapeDtypeStruct((B, DOUT), jnp.float32),
    )(aff_e, mat_e, pa, pm, wa2, wm2, b2d)


def kernel(aff_idx, mat_idx, aff_table, mat_table, W, b):
    ai = aff_idx.astype(jnp.int32)
    mi = mat_idx.astype(jnp.int32)
    aff_idx_r = (ai >> 1).reshape(NW, NCHUNK, CHUNK)
    mat_idx_r = (mi >> 1).reshape(NW, NCHUNK, CHUNK)
    pa = (ai & 1).astype(jnp.float32).reshape(B, 1)
    pm = (mi & 1).astype(jnp.float32).reshape(B, 1)
    aff_pairs = aff_table.reshape(VH, DPAIR)
    mat_pairs = mat_table.reshape(VH, DPAIR)
    aff_e, mat_e = _sc_gather(aff_idx_r, mat_idx_r, aff_pairs, mat_pairs)
    w1t = W[:, :DA].T
    w2t = W[:, DA:].T
    wa2 = jnp.concatenate([w1t, w1t], axis=0)
    wm2 = jnp.concatenate([w2t, w2t], axis=0)
    return _tc_linear(aff_e, mat_e, pa, pm, wa2, wm2, b.reshape(1, DOUT))


# speedup vs baseline: 1.1896x; 1.1896x over previous
import functools

import jax
import jax.numpy as jnp
from jax import lax
from jax.experimental import pallas as pl
from jax.experimental.pallas import tpu as pltpu
from jax.experimental.pallas import tpu_sc as plsc

B = 16384
DA = 64
DM = 64
DOUT = 128
DPAIR = 128
V = 100000
H = 50176
NC = 2
NS = 16
NW = NC * NS
B_PER_W = B // NW
CHUNK = 128
NCHUNK = B_PER_W // CHUNK

_RT = 1024


def _repack_body(lo_ref, hi_ref, out_ref):
    out_ref[...] = jnp.concatenate(
        [lo_ref[...].T, hi_ref[...].T], axis=1)


def _tc_repack(table_t):
    return pl.pallas_call(
        _repack_body,
        grid=(H // _RT,),
        in_specs=[
            pl.BlockSpec((DA, _RT), lambda i: (0, i)),
            pl.BlockSpec((DA, _RT), lambda i: (0, i + H // _RT)),
        ],
        out_specs=pl.BlockSpec((_RT, DPAIR), lambda i: (i, 0)),
        out_shape=jax.ShapeDtypeStruct((H, DPAIR), jnp.float32),
    )(table_t, table_t)


def _sc_gather(idx_r, pairs):
    mesh = plsc.VectorSubcoreMesh(core_axis_name="c", subcore_axis_name="s")

    @functools.partial(
        pl.kernel,
        mesh=mesh,
        out_type=jax.ShapeDtypeStruct((B, DPAIR), jnp.float32),
        scratch_types=[
            pltpu.VMEM((NCHUNK, CHUNK), jnp.int32),
            pltpu.VMEM((B_PER_W, DPAIR), jnp.float32),
            pltpu.SemaphoreType.DMA,
        ],
    )
    def k(idx_hbm, t_hbm, out_hbm, idx_v, rows_v, sem):
        wid = lax.axis_index("s") * NC + lax.axis_index("c")
        base = wid * B_PER_W
        pltpu.sync_copy(idx_hbm.at[wid], idx_v)
        copies = []
        for j in range(NCHUNK):
            copies.append(pltpu.async_copy(
                t_hbm.at[idx_v.at[j]],
                rows_v.at[pl.ds(j * CHUNK, CHUNK)], sem))
        for c in copies:
            c.wait()
        pltpu.sync_copy(rows_v, out_hbm.at[pl.ds(base, B_PER_W)])

    return k(idx_r, pairs)


_BT = 2048


def _tc_body(a_ref, m_ref, pa_ref, pm_ref, wa_ref, wm_ref, b_ref, out_ref):
    lane = lax.broadcasted_iota(jnp.int32, (_BT, DPAIR), 1)
    left = lane < DA
    keep_a = left != (pa_ref[...] > 0.5)
    keep_m = left != (pm_ref[...] > 0.5)
    acc = jnp.dot(jnp.where(keep_a, a_ref[...], 0.0), wa_ref[...],
                  preferred_element_type=jnp.float32)
    acc += jnp.dot(jnp.where(keep_m, m_ref[...], 0.0), wm_ref[...],
                   preferred_element_type=jnp.float32)
    out_ref[...] = jnp.maximum(acc + b_ref[...], 0.0)


def _tc_linear(aff_e, mat_e, pa, pm, wa2, wm2, b2d):
    return pl.pallas_call(
        _tc_body,
        grid=(B // _BT,),
        in_specs=[
            pl.BlockSpec((_BT, DPAIR), lambda i: (i, 0)),
            pl.BlockSpec((_BT, DPAIR), lambda i: (i, 0)),
            pl.BlockSpec((_BT, 1), lambda i: (i, 0)),
            pl.BlockSpec((_BT, 1), lambda i: (i, 0)),
            pl.BlockSpec((DPAIR, DOUT), lambda i: (0, 0)),
            pl.BlockSpec((DPAIR, DOUT), lambda i: (0, 0)),
            pl.BlockSpec((1, DOUT), lambda i: (0, 0)),
        ],
        out_specs=pl.BlockSpec((_BT, DOUT), lambda i: (i, 0)),
        out_shape=jax.ShapeDtypeStruct((B, DOUT), jnp.float32),
    )(aff_e, mat_e, pa, pm, wa2, wm2, b2d)


def kernel(aff_idx, mat_idx, aff_table, mat_table, W, b):
    ai = aff_idx.astype(jnp.int32)
    mi = mat_idx.astype(jnp.int32)
    aff_idx_r = (ai % H).reshape(NW, NCHUNK, CHUNK)
    mat_idx_r = (mi % H).reshape(NW, NCHUNK, CHUNK)
    pa = (ai >= H).astype(jnp.float32).reshape(B, 1)
    pm = (mi >= H).astype(jnp.float32).reshape(B, 1)
    aff_pairs = _tc_repack(aff_table.T)
    aff_e = _sc_gather(aff_idx_r, aff_pairs)
    mat_pairs = _tc_repack(mat_table.T)
    mat_e = _sc_gather(mat_idx_r, mat_pairs)
    w1t = W[:, :DA].T
    w2t = W[:, DA:].T
    wa2 = jnp.concatenate([w1t, w1t], axis=0)
    wm2 = jnp.concatenate([w2t, w2t], axis=0)
    return _tc_linear(aff_e, mat_e, pa, pm, wa2, wm2, b.reshape(1, DOUT))


# speedup vs baseline: 1.2188x; 1.0245x over previous
import functools

import jax
import jax.numpy as jnp
from jax import lax
from jax.experimental import pallas as pl
from jax.experimental.pallas import tpu as pltpu
from jax.experimental.pallas import tpu_sc as plsc

B = 16384
DA = 64
DOUT = 128
V = 100000
NC = 2
NS = 16
NW = NC * NS
B_PER_W = B // NW
CHUNK = 128
NCHUNK = B_PER_W // CHUNK

_RT = 2048


def _proj_body(t_ref, w_ref, b_ref, out_ref):
    out_ref[...] = lax.dot_general(
        t_ref[...], w_ref[...], (((0,), (0,)), ((), ())),
        preferred_element_type=jnp.float32) + b_ref[...]


def _tc_project(table_t, w_half_t, b2d):
    grid = (pl.cdiv(V, _RT),)
    return pl.pallas_call(
        _proj_body,
        grid=grid,
        in_specs=[
            pl.BlockSpec((DA, _RT), lambda i: (0, i)),
            pl.BlockSpec((DA, DOUT), lambda i: (0, 0)),
            pl.BlockSpec((1, DOUT), lambda i: (0, 0)),
        ],
        out_specs=pl.BlockSpec((_RT, DOUT), lambda i: (i, 0)),
        out_shape=jax.ShapeDtypeStruct((V, DOUT), jnp.float32),
    )(table_t, w_half_t, b2d)


def _sc_gather(idx_r, proj):
    mesh = plsc.VectorSubcoreMesh(core_axis_name="c", subcore_axis_name="s")

    @functools.partial(
        pl.kernel,
        mesh=mesh,
        out_type=jax.ShapeDtypeStruct((B, DOUT), jnp.float32),
        scratch_types=[
            pltpu.VMEM((NCHUNK, CHUNK), jnp.int32),
            pltpu.VMEM((B_PER_W, DOUT), jnp.float32),
            pltpu.SemaphoreType.DMA,
        ],
    )
    def k(idx_hbm, t_hbm, out_hbm, idx_v, rows_v, sem):
        wid = lax.axis_index("s") * NC + lax.axis_index("c")
        base = wid * B_PER_W
        pltpu.sync_copy(idx_hbm.at[wid], idx_v)
        copies = []
        for j in range(NCHUNK):
            copies.append(pltpu.async_copy(
                t_hbm.at[idx_v.at[j]],
                rows_v.at[pl.ds(j * CHUNK, CHUNK)], sem))
        for c in copies:
            c.wait()
        pltpu.sync_copy(rows_v, out_hbm.at[pl.ds(base, B_PER_W)])

    return k(idx_r, proj)


_BT = 4096


def _addrelu_body(a_ref, m_ref, out_ref):
    out_ref[...] = jnp.maximum(a_ref[...] + m_ref[...], 0.0)


def _tc_addrelu(ga, gm):
    return pl.pallas_call(
        _addrelu_body,
        grid=(B // _BT,),
        in_specs=[
            pl.BlockSpec((_BT, DOUT), lambda i: (i, 0)),
            pl.BlockSpec((_BT, DOUT), lambda i: (i, 0)),
        ],
        out_specs=pl.BlockSpec((_BT, DOUT), lambda i: (i, 0)),
        out_shape=jax.ShapeDtypeStruct((B, DOUT), jnp.float32),
    )(ga, gm)


def kernel(aff_idx, mat_idx, aff_table, mat_table, W, b):
    aff_idx_r = aff_idx.astype(jnp.int32).reshape(NW, NCHUNK, CHUNK)
    mat_idx_r = mat_idx.astype(jnp.int32).reshape(NW, NCHUNK, CHUNK)
    w1t = W[:, :DA].T
    w2t = W[:, DA:].T
    b2d = b.reshape(1, DOUT)
    zeros = jnp.zeros((1, DOUT), jnp.float32)
    proj_aff = _tc_project(aff_table.T, w1t, b2d)
    ga = _sc_gather(aff_idx_r, proj_aff)
    proj_mat = _tc_project(mat_table.T, w2t, zeros)
    gm = _sc_gather(mat_idx_r, proj_mat)
    return _tc_addrelu(ga, gm)


# speedup vs baseline: 1.5120x; 1.2406x over previous
import functools

import jax
import jax.numpy as jnp
from jax import lax
from jax.experimental import pallas as pl
from jax.experimental.pallas import tpu as pltpu
from jax.experimental.pallas import tpu_sc as plsc

B = 16384
DA = 64
DOUT = 128
V = 100000
NC = 2
NS = 16
NW = NC * NS
B_PER_W = B // NW
CHUNK = 128
NCHUNK = B_PER_W // CHUNK

_RT = 4096


def _proj_body(t_ref, w_ref, b_ref, out_ref):
    out_ref[...] = lax.dot_general(
        t_ref[...], w_ref[...], (((0,), (0,)), ((), ())),
        preferred_element_type=jnp.float32) + b_ref[...]


def _tc_project(table_t, w_half_t, b2d):
    grid = (pl.cdiv(V, _RT),)
    return pl.pallas_call(
        _proj_body,
        grid=grid,
        compiler_params=pltpu.CompilerParams(
            fuse_transposed_lhs_in_matmul=True),
        in_specs=[
            pl.BlockSpec((DA, _RT), lambda i: (0, i)),
            pl.BlockSpec((DA, DOUT), lambda i: (0, 0)),
            pl.BlockSpec((1, DOUT), lambda i: (0, 0)),
        ],
        out_specs=pl.BlockSpec((_RT, DOUT), lambda i: (i, 0)),
        out_shape=jax.ShapeDtypeStruct((V, DOUT), jnp.float32),
    )(table_t, w_half_t, b2d)


def _sc_gather(idx_r, proj):
    mesh = plsc.VectorSubcoreMesh(core_axis_name="c", subcore_axis_name="s")

    @functools.partial(
        pl.kernel,
        mesh=mesh,
        out_type=jax.ShapeDtypeStruct((B, DOUT), jnp.float32),
        scratch_types=[
            pltpu.VMEM((NCHUNK, CHUNK), jnp.int32),
            pltpu.VMEM((B_PER_W, DOUT), jnp.float32),
            pltpu.SemaphoreType.DMA,
        ],
    )
    def k(idx_hbm, t_hbm, out_hbm, idx_v, rows_v, sem):
        wid = lax.axis_index("s") * NC + lax.axis_index("c")
        base = wid * B_PER_W
        pltpu.sync_copy(idx_hbm.at[wid], idx_v)
        copies = []
        for j in range(NCHUNK):
            copies.append(pltpu.async_copy(
                t_hbm.at[idx_v.at[j]],
                rows_v.at[pl.ds(j * CHUNK, CHUNK)], sem))
        for c in copies:
            c.wait()
        pltpu.sync_copy(rows_v, out_hbm.at[pl.ds(base, B_PER_W)])

    return k(idx_r, proj)


_BT = 4096


def _addrelu_body(a_ref, m_ref, out_ref):
    out_ref[...] = jnp.maximum(a_ref[...] + m_ref[...], 0.0)


def _tc_addrelu(ga, gm):
    return pl.pallas_call(
        _addrelu_body,
        grid=(B // _BT,),
        in_specs=[
            pl.BlockSpec((_BT, DOUT), lambda i: (i, 0)),
            pl.BlockSpec((_BT, DOUT), lambda i: (i, 0)),
        ],
        out_specs=pl.BlockSpec((_BT, DOUT), lambda i: (i, 0)),
        out_shape=jax.ShapeDtypeStruct((B, DOUT), jnp.float32),
    )(ga, gm)


def kernel(aff_idx, mat_idx, aff_table, mat_table, W, b):
    aff_idx_r = aff_idx.astype(jnp.int32).reshape(NW, NCHUNK, CHUNK)
    mat_idx_r = mat_idx.astype(jnp.int32).reshape(NW, NCHUNK, CHUNK)
    w1t = W[:, :DA].T
    w2t = W[:, DA:].T
    b2d = b.reshape(1, DOUT)
    zeros = jnp.zeros((1, DOUT), jnp.float32)
    proj_aff = _tc_project(aff_table.T, w1t, b2d)
    ga = _sc_gather(aff_idx_r, proj_aff)
    proj_mat = _tc_project(mat_table.T, w2t, zeros)
    gm = _sc_gather(mat_idx_r, proj_mat)
    return _tc_addrelu(ga, gm)


# speedup vs baseline: 1.5536x; 1.0275x over previous
import functools

import jax
import jax.numpy as jnp
from jax import lax
from jax.experimental import pallas as pl
from jax.experimental.pallas import tpu as pltpu
from jax.experimental.pallas import tpu_sc as plsc

B = 16384
DA = 64
DOUT = 128
V = 100000
NC = 2
NS = 16
NW = NC * NS
B_PER_W = B // NW
CHUNK = 128
NCHUNK = B_PER_W // CHUNK

_RT = 4096


def _proj_body(t_ref, w_ref, b_ref, out_ref):
    out_ref[...] = lax.dot_general(
        t_ref[...].astype(jnp.bfloat16), w_ref[...].astype(jnp.bfloat16),
        (((0,), (0,)), ((), ())),
        preferred_element_type=jnp.float32) + b_ref[...]


def _tc_project(table_t, w_half_t, b2d):
    grid = (pl.cdiv(V, _RT),)
    return pl.pallas_call(
        _proj_body,
        grid=grid,
        compiler_params=pltpu.CompilerParams(
            fuse_transposed_lhs_in_matmul=True),
        in_specs=[
            pl.BlockSpec((DA, _RT), lambda i: (0, i)),
            pl.BlockSpec((DA, DOUT), lambda i: (0, 0)),
            pl.BlockSpec((1, DOUT), lambda i: (0, 0)),
        ],
        out_specs=pl.BlockSpec((_RT, DOUT), lambda i: (i, 0)),
        out_shape=jax.ShapeDtypeStruct((V, DOUT), jnp.float32),
    )(table_t, w_half_t, b2d)


def _sc_gather(idx_r, proj):
    mesh = plsc.VectorSubcoreMesh(core_axis_name="c", subcore_axis_name="s")

    @functools.partial(
        pl.kernel,
        mesh=mesh,
        out_type=jax.ShapeDtypeStruct((B, DOUT), jnp.float32),
        scratch_types=[
            pltpu.VMEM((NCHUNK, CHUNK), jnp.int32),
            pltpu.VMEM((B_PER_W, DOUT), jnp.float32),
            pltpu.SemaphoreType.DMA,
        ],
    )
    def k(idx_hbm, t_hbm, out_hbm, idx_v, rows_v, sem):
        wid = lax.axis_index("s") * NC + lax.axis_index("c")
        base = wid * B_PER_W
        pltpu.sync_copy(idx_hbm.at[wid], idx_v)
        copies = []
        for j in range(NCHUNK):
            copies.append(pltpu.async_copy(
                t_hbm.at[idx_v.at[j]],
                rows_v.at[pl.ds(j * CHUNK, CHUNK)], sem))
        for c in copies:
            c.wait()
        pltpu.sync_copy(rows_v, out_hbm.at[pl.ds(base, B_PER_W)])

    return k(idx_r, proj)


_BT = 4096


def _addrelu_body(a_ref, m_ref, out_ref):
    out_ref[...] = jnp.maximum(a_ref[...] + m_ref[...], 0.0)


def _tc_addrelu(ga, gm):
    return pl.pallas_call(
        _addrelu_body,
        grid=(B // _BT,),
        in_specs=[
            pl.BlockSpec((_BT, DOUT), lambda i: (i, 0)),
            pl.BlockSpec((_BT, DOUT), lambda i: (i, 0)),
        ],
        out_specs=pl.BlockSpec((_BT, DOUT), lambda i: (i, 0)),
        out_shape=jax.ShapeDtypeStruct((B, DOUT), jnp.float32),
    )(ga, gm)


def kernel(aff_idx, mat_idx, aff_table, mat_table, W, b):
    aff_idx_r = aff_idx.astype(jnp.int32).reshape(NW, NCHUNK, CHUNK)
    mat_idx_r = mat_idx.astype(jnp.int32).reshape(NW, NCHUNK, CHUNK)
    w1t = W[:, :DA].T
    w2t = W[:, DA:].T
    b2d = b.reshape(1, DOUT)
    zeros = jnp.zeros((1, DOUT), jnp.float32)
    proj_aff = _tc_project(aff_table.T, w1t, b2d)
    ga = _sc_gather(aff_idx_r, proj_aff)
    proj_mat = _tc_project(mat_table.T, w2t, zeros)
    gm = _sc_gather(mat_idx_r, proj_mat)
    return _tc_addrelu(ga, gm)


# speedup vs baseline: 1.7938x; 1.1546x over previous
import functools

import jax
import jax.numpy as jnp
from jax import lax
from jax.experimental import pallas as pl
from jax.experimental.pallas import tpu as pltpu
from jax.experimental.pallas import tpu_sc as plsc

B = 16384
DA = 64
DOUT = 128
V = 100000
NC = 2
NS = 16
NW = NC * NS
B_PER_W = B // NW
CHUNK = 128
NCHUNK = B_PER_W // CHUNK

_RT = 8192


def _proj_body(t_ref, w_ref, b_ref, out_ref):
    out_ref[...] = lax.dot_general(
        t_ref[...].astype(jnp.bfloat16), w_ref[...].astype(jnp.bfloat16),
        (((0,), (0,)), ((), ())),
        preferred_element_type=jnp.float32) + b_ref[...]


def _tc_project(table_t, w_half_t, b2d):
    grid = (pl.cdiv(V, _RT),)
    return pl.pallas_call(
        _proj_body,
        grid=grid,
        compiler_params=pltpu.CompilerParams(
            fuse_transposed_lhs_in_matmul=True),
        in_specs=[
            pl.BlockSpec((DA, _RT), lambda i: (0, i)),
            pl.BlockSpec((DA, DOUT), lambda i: (0, 0)),
            pl.BlockSpec((1, DOUT), lambda i: (0, 0)),
        ],
        out_specs=pl.BlockSpec((_RT, DOUT), lambda i: (i, 0)),
        out_shape=jax.ShapeDtypeStruct((V, DOUT), jnp.float32),
    )(table_t, w_half_t, b2d)


def _sc_gather(idx_r, proj):
    mesh = plsc.VectorSubcoreMesh(core_axis_name="c", subcore_axis_name="s")

    @functools.partial(
        pl.kernel,
        mesh=mesh,
        out_type=jax.ShapeDtypeStruct((B, DOUT), jnp.float32),
        scratch_types=[
            pltpu.VMEM((NCHUNK, CHUNK), jnp.int32),
            pltpu.VMEM((B_PER_W, DOUT), jnp.float32),
            pltpu.SemaphoreType.DMA,
        ],
    )
    def k(idx_hbm, t_hbm, out_hbm, idx_v, rows_v, sem):
        wid = lax.axis_index("s") * NC + lax.axis_index("c")
        base = wid * B_PER_W
        pltpu.sync_copy(idx_hbm.at[wid], idx_v)
        copies = []
        for j in range(NCHUNK):
            copies.append(pltpu.async_copy(
                t_hbm.at[idx_v.at[j]],
                rows_v.at[pl.ds(j * CHUNK, CHUNK)], sem))
        for c in copies:
            c.wait()
        pltpu.sync_copy(rows_v, out_hbm.at[pl.ds(base, B_PER_W)])

    return k(idx_r, proj)


_BT = 4096


def _addrelu_body(a_ref, m_ref, out_ref):
    out_ref[...] = jnp.maximum(a_ref[...] + m_ref[...], 0.0)


def _tc_addrelu(ga, gm):
    return pl.pallas_call(
        _addrelu_body,
        grid=(B // _BT,),
        in_specs=[
            pl.BlockSpec((_BT, DOUT), lambda i: (i, 0)),
            pl.BlockSpec((_BT, DOUT), lambda i: (i, 0)),
        ],
        out_specs=pl.BlockSpec((_BT, DOUT), lambda i: (i, 0)),
        out_shape=jax.ShapeDtypeStruct((B, DOUT), jnp.float32),
    )(ga, gm)


def kernel(aff_idx, mat_idx, aff_table, mat_table, W, b):
    aff_idx_r = aff_idx.astype(jnp.int32).reshape(NW, NCHUNK, CHUNK)
    mat_idx_r = mat_idx.astype(jnp.int32).reshape(NW, NCHUNK, CHUNK)
    w1t = W[:, :DA].T
    w2t = W[:, DA:].T
    b2d = b.reshape(1, DOUT)
    zeros = jnp.zeros((1, DOUT), jnp.float32)
    proj_aff = _tc_project(aff_table.T, w1t, b2d)
    ga = _sc_gather(aff_idx_r, proj_aff)
    proj_mat = _tc_project(mat_table.T, w2t, zeros)
    gm = _sc_gather(mat_idx_r, proj_mat)
    return _tc_addrelu(ga, gm)


# speedup vs baseline: 1.8679x; 1.0414x over previous
import functools

import jax
import jax.numpy as jnp
from jax import lax
from jax.experimental import pallas as pl
from jax.experimental.pallas import tpu as pltpu
from jax.experimental.pallas import tpu_sc as plsc

B = 16384
DA = 64
DOUT = 128
V = 100000
NC = 2
NS = 16
NW = NC * NS
B_PER_W = B // NW
CHUNK = 128
NCHUNK = B_PER_W // CHUNK

_RT = 16384


def _proj_body(t_ref, w_ref, b_ref, out_ref):
    out_ref[...] = lax.dot_general(
        t_ref[...].astype(jnp.bfloat16), w_ref[...].astype(jnp.bfloat16),
        (((0,), (0,)), ((), ())),
        preferred_element_type=jnp.float32) + b_ref[...]


def _tc_project(table_t, w_half_t, b2d):
    grid = (pl.cdiv(V, _RT),)
    return pl.pallas_call(
        _proj_body,
        grid=grid,
        compiler_params=pltpu.CompilerParams(
            fuse_transposed_lhs_in_matmul=True),
        in_specs=[
            pl.BlockSpec((DA, _RT), lambda i: (0, i)),
            pl.BlockSpec((DA, DOUT), lambda i: (0, 0)),
            pl.BlockSpec((1, DOUT), lambda i: (0, 0)),
        ],
        out_specs=pl.BlockSpec((_RT, DOUT), lambda i: (i, 0)),
        out_shape=jax.ShapeDtypeStruct((V, DOUT), jnp.float32),
    )(table_t, w_half_t, b2d)


def _sc_gather(idx_r, proj):
    mesh = plsc.VectorSubcoreMesh(core_axis_name="c", subcore_axis_name="s")

    @functools.partial(
        pl.kernel,
        mesh=mesh,
        out_type=jax.ShapeDtypeStruct((B, DOUT), jnp.float32),
        scratch_types=[
            pltpu.VMEM((NCHUNK, CHUNK), jnp.int32),
            pltpu.VMEM((B_PER_W, DOUT), jnp.float32),
            pltpu.SemaphoreType.DMA,
        ],
    )
    def k(idx_hbm, t_hbm, out_hbm, idx_v, rows_v, sem):
        wid = lax.axis_index("s") * NC + lax.axis_index("c")
        base = wid * B_PER_W
        pltpu.sync_copy(idx_hbm.at[wid], idx_v)
        copies = []
        for j in range(NCHUNK):
            copies.append(pltpu.async_copy(
                t_hbm.at[idx_v.at[j]],
                rows_v.at[pl.ds(j * CHUNK, CHUNK)], sem))
        for c in copies:
            c.wait()
        pltpu.sync_copy(rows_v, out_hbm.at[pl.ds(base, B_PER_W)])

    return k(idx_r, proj)


_BT = 4096


def _addrelu_body(a_ref, m_ref, out_ref):
    out_ref[...] = jnp.maximum(a_ref[...] + m_ref[...], 0.0)


def _tc_addrelu(ga, gm):
    return pl.pallas_call(
        _addrelu_body,
        grid=(B // _BT,),
        in_specs=[
            pl.BlockSpec((_BT, DOUT), lambda i: (i, 0)),
            pl.BlockSpec((_BT, DOUT), lambda i: (i, 0)),
        ],
        out_specs=pl.BlockSpec((_BT, DOUT), lambda i: (i, 0)),
        out_shape=jax.ShapeDtypeStruct((B, DOUT), jnp.float32),
    )(ga, gm)


def kernel(aff_idx, mat_idx, aff_table, mat_table, W, b):
    aff_idx_r = aff_idx.astype(jnp.int32).reshape(NW, NCHUNK, CHUNK)
    mat_idx_r = mat_idx.astype(jnp.int32).reshape(NW, NCHUNK, CHUNK)
    w1t = W[:, :DA].T
    w2t = W[:, DA:].T
    b2d = b.reshape(1, DOUT)
    zeros = jnp.zeros((1, DOUT), jnp.float32)
    proj_aff = _tc_project(aff_table.T, w1t, b2d)
    ga = _sc_gather(aff_idx_r, proj_aff)
    proj_mat = _tc_project(mat_table.T, w2t, zeros)
    gm = _sc_gather(mat_idx_r, proj_mat)
    return _tc_addrelu(ga, gm)


# speedup vs baseline: 1.9044x; 1.0195x over previous
import functools

import jax
import jax.numpy as jnp
from jax import lax
from jax.experimental import pallas as pl
from jax.experimental.pallas import tpu as pltpu
from jax.experimental.pallas import tpu_sc as plsc

B = 16384
DA = 64
DOUT = 128
V = 100000
NC = 2
NS = 16
NW = NC * NS
B_PER_W = B // NW
CHUNK = 128
NCHUNK = B_PER_W // CHUNK

_RT = 25088


def _proj_body(t_ref, w_ref, b_ref, out_ref):
    out_ref[...] = lax.dot_general(
        t_ref[...].astype(jnp.bfloat16), w_ref[...].astype(jnp.bfloat16),
        (((0,), (0,)), ((), ())),
        preferred_element_type=jnp.float32) + b_ref[...]


def _tc_project(table_t, w_half_t, b2d):
    grid = (pl.cdiv(V, _RT),)
    return pl.pallas_call(
        _proj_body,
        grid=grid,
        compiler_params=pltpu.CompilerParams(
            fuse_transposed_lhs_in_matmul=True),
        in_specs=[
            pl.BlockSpec((DA, _RT), lambda i: (0, i)),
            pl.BlockSpec((DA, DOUT), lambda i: (0, 0)),
            pl.BlockSpec((1, DOUT), lambda i: (0, 0)),
        ],
        out_specs=pl.BlockSpec((_RT, DOUT), lambda i: (i, 0)),
        out_shape=jax.ShapeDtypeStruct((V, DOUT), jnp.float32),
    )(table_t, w_half_t, b2d)


def _sc_gather(idx_r, proj):
    mesh = plsc.VectorSubcoreMesh(core_axis_name="c", subcore_axis_name="s")

    @functools.partial(
        pl.kernel,
        mesh=mesh,
        out_type=jax.ShapeDtypeStruct((B, DOUT), jnp.float32),
        scratch_types=[
            pltpu.VMEM((NCHUNK, CHUNK), jnp.int32),
            pltpu.VMEM((B_PER_W, DOUT), jnp.float32),
            pltpu.SemaphoreType.DMA,
        ],
    )
    def k(idx_hbm, t_hbm, out_hbm, idx_v, rows_v, sem):
        wid = lax.axis_index("s") * NC + lax.axis_index("c")
        base = wid * B_PER_W
        pltpu.sync_copy(idx_hbm.at[wid], idx_v)
        copies = []
        for j in range(NCHUNK):
            copies.append(pltpu.async_copy(
                t_hbm.at[idx_v.at[j]],
                rows_v.at[pl.ds(j * CHUNK, CHUNK)], sem))
        for c in copies:
            c.wait()
        pltpu.sync_copy(rows_v, out_hbm.at[pl.ds(base, B_PER_W)])

    return k(idx_r, proj)


_BT = 4096


def _addrelu_body(a_ref, m_ref, out_ref):
    out_ref[...] = jnp.maximum(a_ref[...] + m_ref[...], 0.0)


def _tc_addrelu(ga, gm):
    return pl.pallas_call(
        _addrelu_body,
        grid=(B // _BT,),
        in_specs=[
            pl.BlockSpec((_BT, DOUT), lambda i: (i, 0)),
            pl.BlockSpec((_BT, DOUT), lambda i: (i, 0)),
        ],
        out_specs=pl.BlockSpec((_BT, DOUT), lambda i: (i, 0)),
        out_shape=jax.ShapeDtypeStruct((B, DOUT), jnp.float32),
    )(ga, gm)


def kernel(aff_idx, mat_idx, aff_table, mat_table, W, b):
    aff_idx_r = aff_idx.astype(jnp.int32).reshape(NW, NCHUNK, CHUNK)
    mat_idx_r = mat_idx.astype(jnp.int32).reshape(NW, NCHUNK, CHUNK)
    w1t = W[:, :DA].T
    w2t = W[:, DA:].T
    b2d = b.reshape(1, DOUT)
    zeros = jnp.zeros((1, DOUT), jnp.float32)
    proj_aff = _tc_project(aff_table.T, w1t, b2d)
    ga = _sc_gather(aff_idx_r, proj_aff)
    proj_mat = _tc_project(mat_table.T, w2t, zeros)
    gm = _sc_gather(mat_idx_r, proj_mat)
    return _tc_addrelu(ga, gm)


# speedup vs baseline: 1.9868x; 1.0433x over previous
import functools

import jax
import jax.numpy as jnp
from jax import lax
from jax.experimental import pallas as pl
from jax.experimental.pallas import tpu as pltpu
from jax.experimental.pallas import tpu_sc as plsc

B = 16384
DA = 64
DOUT = 128
V = 100000
NC = 2
NS = 16
NW = NC * NS
B_PER_W = B // NW
CHUNK = 128
NCHUNK = B_PER_W // CHUNK

_RT = 25088


def _proj_body_bias(t_ref, w_ref, b_ref, out_ref):
    w = w_ref[...][:, :DA].astype(jnp.bfloat16)
    out_ref[...] = lax.dot_general(
        t_ref[...].astype(jnp.bfloat16), w, (((0,), (1,)), ((), ())),
        preferred_element_type=jnp.float32) + b_ref[...]


def _proj_body_nobias(t_ref, w_ref, out_ref):
    w = w_ref[...][:, DA:].astype(jnp.bfloat16)
    out_ref[...] = lax.dot_general(
        t_ref[...].astype(jnp.bfloat16), w, (((0,), (1,)), ((), ())),
        preferred_element_type=jnp.float32)


def _tc_project(table_t, w_full, b2d=None):
    grid = (pl.cdiv(V, _RT),)
    specs = [
        pl.BlockSpec((DA, _RT), lambda i: (0, i)),
        pl.BlockSpec((DOUT, DOUT), lambda i: (0, 0)),
    ]
    args = [table_t, w_full]
    body = _proj_body_nobias
    if b2d is not None:
        specs.append(pl.BlockSpec((1, DOUT), lambda i: (0, 0)))
        args.append(b2d)
        body = _proj_body_bias
    return pl.pallas_call(
        body,
        grid=grid,
        compiler_params=pltpu.CompilerParams(
            fuse_transposed_lhs_in_matmul=True),
        in_specs=specs,
        out_specs=pl.BlockSpec((_RT, DOUT), lambda i: (i, 0)),
        out_shape=jax.ShapeDtypeStruct((V, DOUT), jnp.float32),
    )(*args)


def _sc_gather(idx, proj):
    mesh = plsc.VectorSubcoreMesh(core_axis_name="c", subcore_axis_name="s")

    @functools.partial(
        pl.kernel,
        mesh=mesh,
        out_type=jax.ShapeDtypeStruct((B, DOUT), jnp.float32),
        scratch_types=[
            pltpu.VMEM((B_PER_W,), jnp.int32),
            pltpu.VMEM((B_PER_W, DOUT), jnp.float32),
            pltpu.SemaphoreType.DMA,
        ],
    )
    def k(idx_hbm, t_hbm, out_hbm, idx_v, rows_v, sem):
        wid = lax.axis_index("s") * NC + lax.axis_index("c")
        base = wid * B_PER_W
        pltpu.sync_copy(idx_hbm.at[pl.ds(base, B_PER_W)], idx_v)
        copies = []
        for j in range(NCHUNK):
            copies.append(pltpu.async_copy(
                t_hbm.at[idx_v.at[pl.ds(j * CHUNK, CHUNK)]],
                rows_v.at[pl.ds(j * CHUNK, CHUNK)], sem))
        for c in copies:
            c.wait()
        pltpu.sync_copy(rows_v, out_hbm.at[pl.ds(base, B_PER_W)])

    return k(idx, proj)


_BT = 8192


def _addrelu_body(a_ref, m_ref, out_ref):
    out_ref[...] = jnp.maximum(a_ref[...] + m_ref[...], 0.0)


def _tc_addrelu(ga, gm):
    return pl.pallas_call(
        _addrelu_body,
        grid=(B // _BT,),
        in_specs=[
            pl.BlockSpec((_BT, DOUT), lambda i: (i, 0)),
            pl.BlockSpec((_BT, DOUT), lambda i: (i, 0)),
        ],
        out_specs=pl.BlockSpec((_BT, DOUT), lambda i: (i, 0)),
        out_shape=jax.ShapeDtypeStruct((B, DOUT), jnp.float32),
    )(ga, gm)


def kernel(aff_idx, mat_idx, aff_table, mat_table, W, b):
    ai = aff_idx.astype(jnp.int32)
    mi = mat_idx.astype(jnp.int32)
    b2d = b.reshape(1, DOUT)
    proj_aff = _tc_project(aff_table.T, W, b2d)
    ga = _sc_gather(ai, proj_aff)
    proj_mat = _tc_project(mat_table.T, W)
    gm = _sc_gather(mi, proj_mat)
    return _tc_addrelu(ga, gm)
